# baseline (speedup 1.0000x reference)
import jax, jax.numpy as jnp
from jax.experimental import pallas as pl


def _copy_k(x_ref, o_ref):
    o_ref[...] = x_ref[...]


def kernel(vertices, faces, face_edges, face_edges_mask, coor_embed,
           W_in, b_in, Wl0, Wr0, bl0, Wl1, Wr1, bl1):
    b, nf = faces.shape[0], faces.shape[1]
    batch_idx = jnp.arange(b)[:, None, None]
    face_coords = vertices[batch_idx, faces].reshape(b, nf, 9)
    fe = coor_embed[face_coords].reshape(b, nf, 9 * coor_embed.shape[-1])
    fe = fe @ W_in.T + b_in
    x = fe.reshape(b * nf, -1)
    pad_node_id = b * nf
    x = jnp.concatenate([x, jnp.zeros((1, x.shape[-1]), x.dtype)], axis=0)
    offs = (jnp.arange(b) * nf)[:, None, None]
    edges = face_edges + offs
    edges = jnp.where(face_edges_mask[:, None, :], edges, pad_node_id)
    edges = edges.transpose(1, 0, 2).reshape(2, -1)
    src, dst = edges[0], edges[1]
    N = b * nf + 1
    def sage_conv(h, Wl, Wr, bias):
        msgs = h[src]
        sums = jax.ops.segment_sum(msgs, dst, num_segments=N)
        cnt = jax.ops.segment_sum(jnp.ones((src.shape[0],), h.dtype), dst, num_segments=N)
        mean = sums / jnp.clip(cnt, 1.0, None)[:, None]
        return mean @ Wl.T + h @ Wr.T + bias
    x = sage_conv(x, Wl0, Wr0, bl0)
    x = sage_conv(x, Wl1, Wr1, bl1)
    xp = jnp.pad(x, ((0, 100352 - x.shape[0]), (0, 0)))
    xp = pl.pallas_call(
        _copy_k,
        grid=(100352 // 2048,),
        in_specs=[pl.BlockSpec((2048, 128), lambda i: (i, 0))],
        out_specs=pl.BlockSpec((2048, 128), lambda i: (i, 0)),
        out_shape=jax.ShapeDtypeStruct(xp.shape, xp.dtype),
    )(xp)
    return xp[: b * nf].reshape(b, nf, -1)


# speedup vs baseline: 3.3279x; 3.3279x over previous
import functools
import jax
import jax.numpy as jnp
from jax import lax
from jax.experimental import pallas as pl
from jax.experimental.pallas import tpu as pltpu
from jax.experimental.pallas import tpu_sc as plsc

B, NV, NF, E = 2, 25000, 50000, 150000
DIM = 128
DCE = 64
NDC = 128

BN = B * NV
BNp = 50176
NFACE = B * NF
NP = 102400
VBLK = 512
NW = 32


def _k0_body(ce_ref, w_ref, t_ref):
    t_ref[0] = jax.lax.dot_general(
        ce_ref[...], w_ref[0], (((1,), (1,)), ((), ())),
        preferred_element_type=jnp.float32)


def _fuse_tables(coor_embed, W_in):
    w9 = W_in.reshape(DIM, 9, DCE).transpose(1, 0, 2)
    return pl.pallas_call(
        _k0_body,
        grid=(9,),
        in_specs=[
            pl.BlockSpec((NDC, DCE), lambda i: (0, 0)),
            pl.BlockSpec((1, DIM, DCE), lambda i: (i, 0, 0)),
        ],
        out_specs=pl.BlockSpec((1, NDC, DIM), lambda i: (i, 0, 0)),
        out_shape=jax.ShapeDtypeStruct((9, NDC, DIM), jnp.float32),
    )(coor_embed, w9)


def _k1_body(c0_ref, c1_ref, c2_ref, t_ref, u_ref):
    acc = jnp.zeros((VBLK, DIM), jnp.float32)
    for c, cref in enumerate((c0_ref, c1_ref, c2_ref)):
        code = cref[0, 0, :]
        onehot_t = (lax.broadcasted_iota(jnp.int32, (NDC, VBLK), 0)
                    == code[None, :]).astype(jnp.float32)
        acc = acc + jax.lax.dot_general(
            onehot_t, t_ref[c], (((0,), (0,)), ((), ())),
            preferred_element_type=jnp.float32)
    u_ref[0] = acc


def _vertex_embed(codes, tables):
    nblk = BNp // VBLK
    return pl.pallas_call(
        _k1_body,
        grid=(3, nblk),
        in_specs=[
            pl.BlockSpec((1, 1, VBLK), lambda s, i: (i, 0, 0)),
            pl.BlockSpec((1, 1, VBLK), lambda s, i: (i, 0, 0)),
            pl.BlockSpec((1, 1, VBLK), lambda s, i: (i, 0, 0)),
            pl.BlockSpec((3, NDC, DIM), lambda s, i: (s, 0, 0)),
        ],
        out_specs=pl.BlockSpec((1, VBLK, DIM), lambda s, i: (s, i, 0)),
        out_shape=jax.ShapeDtypeStruct((3, BNp, DIM), jnp.float32),
    )(codes[0], codes[1], codes[2], tables)


FB = 128
FCHUNK = NP // NW
NB2 = FCHUNK // FB


def _k2_body(u_hbm, f0_hbm, f1_hbm, f2_hbm, bias_hbm, x_hbm,
             f0_v, f1_v, f2_v, b0, b1, b2, outb, bias_v,
             sem0, sem1, sem2):
    wid = lax.axis_index("s") * 2 + lax.axis_index("c")
    base = wid * FCHUNK
    pltpu.sync_copy(f0_hbm.at[pl.ds(base, FCHUNK)], f0_v)
    pltpu.sync_copy(f1_hbm.at[pl.ds(base, FCHUNK)], f1_v)
    pltpu.sync_copy(f2_hbm.at[pl.ds(base, FCHUNK)], f2_v)
    pltpu.sync_copy(bias_hbm, bias_v)

    def batch(j, carry):
        off = j * FB
        cp0 = pltpu.async_copy(u_hbm.at[f0_v.at[pl.ds(off, FB)]], b0, sem0)
        cp1 = pltpu.async_copy(u_hbm.at[f1_v.at[pl.ds(off, FB)]], b1, sem1)
        cp2 = pltpu.async_copy(u_hbm.at[f2_v.at[pl.ds(off, FB)]], b2, sem2)
        cp0.wait()
        cp1.wait()
        cp2.wait()

        def row(r, carry2):
            for k in range(DIM // 16):
                sl = pl.ds(k * 16, 16)
                outb[r, sl] = (b0[r, sl] + b1[r, sl] + b2[r, sl]
                               + bias_v[sl])
            return carry2
        lax.fori_loop(0, FB, row, 0)
        pltpu.sync_copy(outb, x_hbm.at[pl.ds(base + off, FB)])
        return carry
    lax.fori_loop(0, NB2, batch, 0)


def _face_assemble(u_flat, fidx, b_in):
    mesh = plsc.VectorSubcoreMesh(core_axis_name="c", subcore_axis_name="s")
    kf = pl.kernel(
        _k2_body,
        out_type=jax.ShapeDtypeStruct((NP, DIM), jnp.float32),
        mesh=mesh,
        scratch_types=[
            pltpu.VMEM((FCHUNK,), jnp.int32),
            pltpu.VMEM((FCHUNK,), jnp.int32),
            pltpu.VMEM((FCHUNK,), jnp.int32),
            pltpu.VMEM((FB, DIM), jnp.float32),
            pltpu.VMEM((FB, DIM), jnp.float32),
            pltpu.VMEM((FB, DIM), jnp.float32),
            pltpu.VMEM((FB, DIM), jnp.float32),
            pltpu.VMEM((DIM,), jnp.float32),
            pltpu.SemaphoreType.DMA,
            pltpu.SemaphoreType.DMA,
            pltpu.SemaphoreType.DMA,
        ],
    )
    return kf(u_flat, fidx[0], fidx[1], fidx[2], b_in)


EP = 300032
SEG = 4688
NSEGF = EP // SEG
NVREG = SEG // 16
EB = 64
TRANGE = NP // NW
PBUCK = 10
BUCK = TRANGE // PBUCK
ACCR = BUCK + 8
CMPTRASH = SEG + EB
LCAP = EP + SEG + 16


def _k3a_body(src_hbm, dst_hbm, lsrc_hbm, ldst_hbm, lens_hbm,
              seg_src, seg_dst, cmp_src, cmp_dst, cur_v, sem):
    wid = lax.axis_index("s") * 2 + lax.axis_index("c")
    lo = wid * TRANGE

    def seg_body(g, cur):
        pltpu.sync_copy(src_hbm.at[pl.ds(pl.multiple_of(g * SEG, 16), SEG)], seg_src)
        pltpu.sync_copy(dst_hbm.at[pl.ds(pl.multiple_of(g * SEG, 16), SEG)], seg_dst)

        def cvreg(v, cnt_v):
            d = seg_dst[pl.ds(v * 16, 16)]
            s = seg_src[pl.ds(v * 16, 16)]
            m = (d >= lo) & (d < lo + TRANGE)
            pc = plsc.cumsum(m.astype(jnp.int32))
            pos = jnp.where(m, cnt_v + pc - 1, CMPTRASH)
            plsc.store_scatter(cmp_src, [pos], s)
            plsc.store_scatter(cmp_dst, [pos], d)
            return cnt_v + plsc.all_reduce_population_count(m)
        cnt_v = lax.fori_loop(0, NVREG, cvreg, jnp.zeros((16,), jnp.int32))
        cur_v[pl.ds(0, 16)] = cnt_v
        cnt_c = cur_v[pl.ds(0, 16)][0]
        cnt16 = ((cnt_c + 15) // 16) * 16

        def pad16(j, c):
            cmp_src[pl.ds(cnt16 + j * 16, 16)] = jnp.zeros((16,), jnp.int32)
            cmp_dst[pl.ds(cnt16 + j * 16, 16)] = jnp.full((16,), NP,
                                                          jnp.int32)
            return c
        lax.fori_loop(0, (SEG - cnt16) // 16 + 1, pad16, 0)
        cmp_dst[pl.ds(cnt_c, 16)] = jnp.full((16,), NP, jnp.int32)

        pltpu.sync_copy(cmp_src.at[pl.ds(0, SEG)],
                        lsrc_hbm.at[pl.ds(pl.multiple_of(wid * LCAP + cur, 16), SEG)])
        pltpu.sync_copy(cmp_dst.at[pl.ds(0, SEG)],
                        ldst_hbm.at[pl.ds(pl.multiple_of(wid * LCAP + cur, 16), SEG)])
        return cur + cnt16
    cur = lax.fori_loop(0, NSEGF, seg_body, jnp.int32(0))

    def inert(v, c):
        cmp_dst[pl.ds(v * 16, 16)] = jnp.full((16,), NP, jnp.int32)
        return c
    lax.fori_loop(0, NVREG, inert, 0)
    pltpu.sync_copy(cmp_dst.at[pl.ds(0, SEG)],
                    ldst_hbm.at[pl.ds(pl.multiple_of(wid * LCAP + cur, 16), SEG)])
    cur_v[pl.ds(0, 16)] = jnp.zeros((16,), jnp.int32) + cur
    pltpu.sync_copy(cur_v, lens_hbm.at[pl.ds(pl.multiple_of(wid * 16, 16), 16)])


def _partition(src, dst):
    mesh = plsc.VectorSubcoreMesh(core_axis_name="c", subcore_axis_name="s")
    kf = pl.kernel(
        _k3a_body,
        out_type=(jax.ShapeDtypeStruct((NW * LCAP,), jnp.int32),
                  jax.ShapeDtypeStruct((NW * LCAP,), jnp.int32),
                  jax.ShapeDtypeStruct((NW * 16,), jnp.int32)),
        mesh=mesh,
        scratch_types=[
            pltpu.VMEM((SEG,), jnp.int32),
            pltpu.VMEM((SEG,), jnp.int32),
            pltpu.VMEM((SEG + EB + 32,), jnp.int32),
            pltpu.VMEM((SEG + EB + 32,), jnp.int32),
            pltpu.VMEM((16,), jnp.int32),
            pltpu.SemaphoreType.DMA,
        ],
        compiler_params=pltpu.CompilerParams(needs_layout_passes=False),
    )
    return kf(src, dst)


def _k3b_body(h_hbm, lsrc_hbm, ldst_hbm, lens_hbm, agg_hbm, cnt_hbm,
              seg_src, seg_dst, cmp_src, cmp_off, rows_v, len_v, acc,
              acc_cnt, sem):
    wid = lax.axis_index("s") * 2 + lax.axis_index("c")
    base = wid * LCAP
    pltpu.sync_copy(lens_hbm.at[pl.ds(pl.multiple_of(wid * 16, 16), 16)], len_v)
    mylen = len_v[pl.ds(0, 16)][0]
    nseg = (mylen + SEG - 1) // SEG

    def bucket(p, c0):
        lo = wid * TRANGE + p * BUCK

        def zacc(i, c):
            for k in range(DIM // 16):
                acc[i, pl.ds(k * 16, 16)] = jnp.zeros((16,), jnp.float32)
            acc_cnt[i, :] = jnp.zeros((16,), jnp.float32)
            return c
        lax.fori_loop(0, ACCR, zacc, 0)

        def seg_body(g, c1):
            pltpu.sync_copy(lsrc_hbm.at[pl.ds(pl.multiple_of(base + g * SEG, 16), SEG)], seg_src)
            pltpu.sync_copy(ldst_hbm.at[pl.ds(pl.multiple_of(base + g * SEG, 16), SEG)], seg_dst)

            def cvreg(v, cnt_v):
                d = seg_dst[pl.ds(v * 16, 16)]
                s = seg_src[pl.ds(v * 16, 16)]
                m = (d >= lo) & (d < lo + BUCK)
                pc = plsc.cumsum(m.astype(jnp.int32))
                pos = jnp.where(m, cnt_v + pc - 1, CMPTRASH)
                plsc.store_scatter(cmp_src, [pos], s)
                plsc.store_scatter(cmp_off, [pos], d - lo)
                return cnt_v + plsc.all_reduce_population_count(m)
            cnt_v = lax.fori_loop(0, NVREG, cvreg,
                                  jnp.zeros((16,), jnp.int32))
            cmp_off[pl.ds(SEG + EB, 16)] = cnt_v
            cnt_c = cmp_off[pl.ds(SEG + EB, 16)][0]

            for j in range(EB // 16):
                cmp_src[pl.ds(cnt_c + j * 16, 16)] = jnp.zeros((16,),
                                                               jnp.int32)
                cmp_off[pl.ds(cnt_c + j * 16, 16)] = jnp.full((16,), BUCK,
                                                              jnp.int32)
            nb = (cnt_c + EB - 1) // EB

            def batch(j, c):
                o = j * EB
                pltpu.async_copy(h_hbm.at[cmp_src.at[pl.ds(o, EB)]], rows_v,
                                 sem).wait()

                def edge(r, c2):
                    off = cmp_off[pl.ds(o + r, 16)][0]
                    for k in range(DIM // 16):
                        sl = pl.ds(k * 16, 16)
                        acc[off, sl] = acc[off, sl] + rows_v[r, sl]
                    acc_cnt[off, :] = (acc_cnt[off, :]
                                       + jnp.full((16,), 1.0, jnp.float32))
                    return c2
                lax.fori_loop(0, EB, edge, 0)
                return c
            lax.fori_loop(0, nb, batch, 0)
            return c1
        lax.fori_loop(0, nseg, seg_body, 0)

        pltpu.sync_copy(acc.at[pl.ds(0, BUCK)], agg_hbm.at[pl.ds(lo, BUCK)])
        pltpu.sync_copy(acc_cnt.at[pl.ds(0, BUCK)],
                        cnt_hbm.at[pl.ds(lo, BUCK)])
        return c0
    lax.fori_loop(0, PBUCK, bucket, 0)


def _segment_sum(h, lsrc, ldst, lens):
    mesh = plsc.VectorSubcoreMesh(core_axis_name="c", subcore_axis_name="s")
    kf = pl.kernel(
        _k3b_body,
        out_type=(jax.ShapeDtypeStruct((NP, DIM), jnp.float32),
                  jax.ShapeDtypeStruct((NP, 16), jnp.float32)),
        mesh=mesh,
        scratch_types=[
            pltpu.VMEM((SEG,), jnp.int32),
            pltpu.VMEM((SEG,), jnp.int32),
            pltpu.VMEM((SEG + EB + 32,), jnp.int32),
            pltpu.VMEM((SEG + EB + 32,), jnp.int32),
            pltpu.VMEM((EB, DIM), jnp.float32),
            pltpu.VMEM((16,), jnp.int32),
            pltpu.VMEM((ACCR, DIM), jnp.float32),
            pltpu.VMEM((ACCR, 16), jnp.float32),
            pltpu.SemaphoreType.DMA,
        ],
        compiler_params=pltpu.CompilerParams(needs_layout_passes=False),
    )
    return kf(h, lsrc, ldst, lens)


KBLK = 512


def _k4_body(agg_ref, h_ref, cnt_ref, wl_ref, wr_ref, bias_ref, o_ref):
    i = pl.program_id(0)
    inv = 1.0 / jnp.maximum(cnt_ref[:, 0:1], 1.0)
    mean = agg_ref[...] * inv
    out = (jnp.dot(mean, wl_ref[...], preferred_element_type=jnp.float32)
           + jnp.dot(h_ref[...], wr_ref[...], preferred_element_type=jnp.float32)
           + bias_ref[...])
    rows = i * KBLK + lax.broadcasted_iota(jnp.int32, (KBLK, DIM), 0)
    o_ref[...] = jnp.where(rows < NFACE, out, 0.0)


def _dense(agg, h, cnt, wl_t, wr_t, bias):
    return pl.pallas_call(
        _k4_body,
        grid=(NP // KBLK,),
        in_specs=[
            pl.BlockSpec((KBLK, DIM), lambda i: (i, 0)),
            pl.BlockSpec((KBLK, DIM), lambda i: (i, 0)),
            pl.BlockSpec((KBLK, 16), lambda i: (i, 0)),
            pl.BlockSpec((DIM, DIM), lambda i: (0, 0)),
            pl.BlockSpec((DIM, DIM), lambda i: (0, 0)),
            pl.BlockSpec((1, DIM), lambda i: (0, 0)),
        ],
        out_specs=pl.BlockSpec((KBLK, DIM), lambda i: (i, 0)),
        out_shape=jax.ShapeDtypeStruct((NP, DIM), jnp.float32),
    )(agg, h, cnt, wl_t, wr_t, bias)


def kernel(vertices, faces, face_edges, face_edges_mask, coor_embed,
           W_in, b_in, Wl0, Wr0, bl0, Wl1, Wr1, bl1):
    codes = []
    for c in range(3):
        cc = vertices[:, :, c].reshape(BN)
        cc = jnp.pad(cc, (0, BNp - BN))
        codes.append(cc.reshape(BNp // VBLK, 1, VBLK))
    vert_off = jnp.repeat(jnp.arange(B, dtype=jnp.int32) * NV, NF)
    fidx = []
    for s in range(3):
        fs = faces[:, :, s].reshape(NFACE) + vert_off + s * BNp
        fs = jnp.pad(fs, (0, NP - NFACE), constant_values=s * BNp)
        fidx.append(fs)

    tables = _fuse_tables(coor_embed, W_in)
    u = _vertex_embed(codes, tables)
    u_flat = u.reshape(3 * BNp, DIM)
    x = _face_assemble(u_flat, fidx, b_in)

    eoff = (jnp.arange(B, dtype=jnp.int32) * NF)[:, None, None]
    edges = face_edges + eoff
    src = jnp.pad(edges[:, 0, :].reshape(B * E), (0, EP - B * E),
                  constant_values=NFACE)
    dst = jnp.pad(edges[:, 1, :].reshape(B * E), (0, EP - B * E),
                  constant_values=NFACE)

    lsrc, ldst, lens = _partition(src, dst)
    agg0, cnt = _segment_sum(x, lsrc, ldst, lens)
    h1 = _dense(agg0, x, cnt, Wl0.T, Wr0.T, bl0.reshape(1, DIM))
    agg1, _ = _segment_sum(h1, lsrc, ldst, lens)
    h2 = _dense(agg1, h1, cnt, Wl1.T, Wr1.T, bl1.reshape(1, DIM))
    return h2[:NFACE].reshape(B, NF, DIM)


# speedup vs baseline: 3.3290x; 1.0003x over previous
import jax
import jax.numpy as jnp
from jax import lax
from jax.experimental import pallas as pl
from jax.experimental.pallas import tpu as pltpu
from jax.experimental.pallas import tpu_sc as plsc

B, NV, NF, E = 2, 25000, 50000, 150000
DIM = 128
DCE = 64
NDC = 128

BN = B * NV
BNp = 50176
NFACE = B * NF
NP = 102400
VBLK = 512
NW = 32


def _k0_body(ce_ref, w_ref, t_ref):
    t_ref[0] = jax.lax.dot_general(
        ce_ref[...], w_ref[0], (((1,), (1,)), ((), ())),
        preferred_element_type=jnp.float32)


def _fuse_tables(coor_embed, W_in):
    w9 = W_in.reshape(DIM, 9, DCE).transpose(1, 0, 2)
    return pl.pallas_call(
        _k0_body,
        grid=(9,),
        in_specs=[
            pl.BlockSpec((NDC, DCE), lambda i: (0, 0)),
            pl.BlockSpec((1, DIM, DCE), lambda i: (i, 0, 0)),
        ],
        out_specs=pl.BlockSpec((1, NDC, DIM), lambda i: (i, 0, 0)),
        out_shape=jax.ShapeDtypeStruct((9, NDC, DIM), jnp.float32),
    )(coor_embed, w9)


def _k1_body(c0_ref, c1_ref, c2_ref, t_ref, u_ref):
    acc = jnp.zeros((VBLK, DIM), jnp.float32)
    for c, cref in enumerate((c0_ref, c1_ref, c2_ref)):
        code = cref[0, 0, :]
        onehot_t = (lax.broadcasted_iota(jnp.int32, (NDC, VBLK), 0)
                    == code[None, :]).astype(jnp.float32)
        acc = acc + jax.lax.dot_general(
            onehot_t, t_ref[c], (((0,), (0,)), ((), ())),
            preferred_element_type=jnp.float32)
    u_ref[0] = acc


def _vertex_embed(codes, tables):
    nblk = BNp // VBLK
    return pl.pallas_call(
        _k1_body,
        grid=(3, nblk),
        in_specs=[
            pl.BlockSpec((1, 1, VBLK), lambda s, i: (i, 0, 0)),
            pl.BlockSpec((1, 1, VBLK), lambda s, i: (i, 0, 0)),
            pl.BlockSpec((1, 1, VBLK), lambda s, i: (i, 0, 0)),
            pl.BlockSpec((3, NDC, DIM), lambda s, i: (s, 0, 0)),
        ],
        out_specs=pl.BlockSpec((1, VBLK, DIM), lambda s, i: (s, i, 0)),
        out_shape=jax.ShapeDtypeStruct((3, BNp, DIM), jnp.float32),
    )(codes[0], codes[1], codes[2], tables)


FB = 128
FCHUNK = NP // NW
NB2 = FCHUNK // FB


def _k2_body(u_hbm, f0_hbm, f1_hbm, f2_hbm, bias_hbm, x_hbm,
             f0_v, f1_v, f2_v, b0, b1, b2, outb, bias_v,
             sem0, sem1, sem2):
    wid = lax.axis_index("s") * 2 + lax.axis_index("c")
    base = wid * FCHUNK
    pltpu.sync_copy(f0_hbm.at[pl.ds(base, FCHUNK)], f0_v)
    pltpu.sync_copy(f1_hbm.at[pl.ds(base, FCHUNK)], f1_v)
    pltpu.sync_copy(f2_hbm.at[pl.ds(base, FCHUNK)], f2_v)
    pltpu.sync_copy(bias_hbm, bias_v)

    def batch(j, carry):
        off = j * FB
        cp0 = pltpu.async_copy(u_hbm.at[f0_v.at[pl.ds(off, FB)]], b0, sem0)
        cp1 = pltpu.async_copy(u_hbm.at[f1_v.at[pl.ds(off, FB)]], b1, sem1)
        cp2 = pltpu.async_copy(u_hbm.at[f2_v.at[pl.ds(off, FB)]], b2, sem2)
        cp0.wait()
        cp1.wait()
        cp2.wait()

        def row(r, carry2):
            for k in range(DIM // 16):
                sl = pl.ds(k * 16, 16)
                outb[r, sl] = (b0[r, sl] + b1[r, sl] + b2[r, sl]
                               + bias_v[sl])
            return carry2
        lax.fori_loop(0, FB, row, 0)
        pltpu.sync_copy(outb, x_hbm.at[pl.ds(base + off, FB)])
        return carry
    lax.fori_loop(0, NB2, batch, 0)


def _face_assemble(u_flat, fidx, b_in):
    mesh = plsc.VectorSubcoreMesh(core_axis_name="c", subcore_axis_name="s")
    kf = pl.kernel(
        _k2_body,
        out_type=jax.ShapeDtypeStruct((NP, DIM), jnp.float32),
        mesh=mesh,
        scratch_types=[
            pltpu.VMEM((FCHUNK,), jnp.int32),
            pltpu.VMEM((FCHUNK,), jnp.int32),
            pltpu.VMEM((FCHUNK,), jnp.int32),
            pltpu.VMEM((FB, DIM), jnp.float32),
            pltpu.VMEM((FB, DIM), jnp.float32),
            pltpu.VMEM((FB, DIM), jnp.float32),
            pltpu.VMEM((FB, DIM), jnp.float32),
            pltpu.VMEM((DIM,), jnp.float32),
            pltpu.SemaphoreType.DMA,
            pltpu.SemaphoreType.DMA,
            pltpu.SemaphoreType.DMA,
        ],
    )
    return kf(u_flat, fidx[0], fidx[1], fidx[2], b_in)


EP = 300032
SEG = 4688
NSEGF = EP // SEG
NVREG = SEG // 16
EB = 64
TRANGE = NP // NW
PBUCK = 10
BUCK = TRANGE // PBUCK
ACCR = BUCK + 8
CMPTRASH = SEG + EB
LCAP = EP + SEG + 16


def _k3a_body(src_hbm, dst_hbm, lsrc_hbm, ldst_hbm, lens_hbm,
              seg_src, seg_dst, cmp_src, cmp_dst, cur_v, sem):
    wid = lax.axis_index("s") * 2 + lax.axis_index("c")
    lo = wid * TRANGE

    def seg_body(g, cur):
        pltpu.sync_copy(src_hbm.at[pl.ds(pl.multiple_of(g * SEG, 16), SEG)], seg_src)
        pltpu.sync_copy(dst_hbm.at[pl.ds(pl.multiple_of(g * SEG, 16), SEG)], seg_dst)

        def cvreg(v, cnt_v):
            d = seg_dst[pl.ds(v * 16, 16)]
            s = seg_src[pl.ds(v * 16, 16)]
            m = (d >= lo) & (d < lo + TRANGE)
            pc = plsc.cumsum(m.astype(jnp.int32))
            pos = jnp.where(m, cnt_v + pc - 1, CMPTRASH)
            plsc.store_scatter(cmp_src, [pos], s)
            plsc.store_scatter(cmp_dst, [pos], d)
            return cnt_v + plsc.all_reduce_population_count(m)
        cnt_v = lax.fori_loop(0, NVREG, cvreg, jnp.zeros((16,), jnp.int32))
        cur_v[pl.ds(0, 16)] = cnt_v
        cnt_c = cur_v[pl.ds(0, 16)][0]
        cnt16 = ((cnt_c + 15) // 16) * 16

        def pad16(j, c):
            cmp_src[pl.ds(cnt16 + j * 16, 16)] = jnp.zeros((16,), jnp.int32)
            cmp_dst[pl.ds(cnt16 + j * 16, 16)] = jnp.full((16,), NP,
                                                          jnp.int32)
            return c
        lax.fori_loop(0, (SEG - cnt16) // 16 + 1, pad16, 0)
        cmp_dst[pl.ds(cnt_c, 16)] = jnp.full((16,), NP, jnp.int32)

        pltpu.sync_copy(cmp_src.at[pl.ds(0, SEG)],
                        lsrc_hbm.at[pl.ds(pl.multiple_of(wid * LCAP + cur, 16), SEG)])
        pltpu.sync_copy(cmp_dst.at[pl.ds(0, SEG)],
                        ldst_hbm.at[pl.ds(pl.multiple_of(wid * LCAP + cur, 16), SEG)])
        return cur + cnt16
    cur = lax.fori_loop(0, NSEGF, seg_body, jnp.int32(0))

    def inert(v, c):
        cmp_dst[pl.ds(v * 16, 16)] = jnp.full((16,), NP, jnp.int32)
        return c
    lax.fori_loop(0, NVREG, inert, 0)
    pltpu.sync_copy(cmp_dst.at[pl.ds(0, SEG)],
                    ldst_hbm.at[pl.ds(pl.multiple_of(wid * LCAP + cur, 16), SEG)])
    cur_v[pl.ds(0, 16)] = jnp.zeros((16,), jnp.int32) + cur
    pltpu.sync_copy(cur_v, lens_hbm.at[pl.ds(pl.multiple_of(wid * 16, 16), 16)])


def _partition(src, dst):
    mesh = plsc.VectorSubcoreMesh(core_axis_name="c", subcore_axis_name="s")
    kf = pl.kernel(
        _k3a_body,
        out_type=(jax.ShapeDtypeStruct((NW * LCAP,), jnp.int32),
                  jax.ShapeDtypeStruct((NW * LCAP,), jnp.int32),
                  jax.ShapeDtypeStruct((NW * 16,), jnp.int32)),
        mesh=mesh,
        scratch_types=[
            pltpu.VMEM((SEG,), jnp.int32),
            pltpu.VMEM((SEG,), jnp.int32),
            pltpu.VMEM((SEG + EB + 32,), jnp.int32),
            pltpu.VMEM((SEG + EB + 32,), jnp.int32),
            pltpu.VMEM((16,), jnp.int32),
            pltpu.SemaphoreType.DMA,
        ],
        compiler_params=pltpu.CompilerParams(needs_layout_passes=False),
    )
    return kf(src, dst)


def _k3b_body(h_hbm, lsrc_hbm, ldst_hbm, lens_hbm, agg_hbm, cnt_hbm,
              seg_src, seg_dst, cmp_src, cmp_off, rows_v, len_v, acc,
              acc_cnt, sem):
    wid = lax.axis_index("s") * 2 + lax.axis_index("c")
    base = wid * LCAP
    pltpu.sync_copy(lens_hbm.at[pl.ds(pl.multiple_of(wid * 16, 16), 16)], len_v)
    mylen = len_v[pl.ds(0, 16)][0]
    nseg = (mylen + SEG - 1) // SEG

    def bucket(p, c0):
        lo = wid * TRANGE + p * BUCK

        def zacc(i, c):
            for k in range(DIM // 16):
                acc[i, pl.ds(k * 16, 16)] = jnp.zeros((16,), jnp.float32)
            acc_cnt[i, :] = jnp.zeros((16,), jnp.float32)
            return c
        lax.fori_loop(0, ACCR, zacc, 0)

        def seg_body(g, c1):
            pltpu.sync_copy(lsrc_hbm.at[pl.ds(pl.multiple_of(base + g * SEG, 16), SEG)], seg_src)
            pltpu.sync_copy(ldst_hbm.at[pl.ds(pl.multiple_of(base + g * SEG, 16), SEG)], seg_dst)

            def cvreg(v, cnt_v):
                d = seg_dst[pl.ds(v * 16, 16)]
                s = seg_src[pl.ds(v * 16, 16)]
                m = (d >= lo) & (d < lo + BUCK)
                pc = plsc.cumsum(m.astype(jnp.int32))
                pos = jnp.where(m, cnt_v + pc - 1, CMPTRASH)
                plsc.store_scatter(cmp_src, [pos], s)
                plsc.store_scatter(cmp_off, [pos], d - lo)
                return cnt_v + plsc.all_reduce_population_count(m)
            cnt_v = lax.fori_loop(0, NVREG, cvreg,
                                  jnp.zeros((16,), jnp.int32))
            cmp_off[pl.ds(SEG + EB, 16)] = cnt_v
            cnt_c = cmp_off[pl.ds(SEG + EB, 16)][0]

            for j in range(EB // 16):
                cmp_src[pl.ds(cnt_c + j * 16, 16)] = jnp.zeros((16,),
                                                               jnp.int32)
                cmp_off[pl.ds(cnt_c + j * 16, 16)] = jnp.full((16,), BUCK,
                                                              jnp.int32)
            nb = (cnt_c + EB - 1) // EB

            def batch(j, c):
                o = j * EB
                pltpu.async_copy(h_hbm.at[cmp_src.at[pl.ds(o, EB)]], rows_v,
                                 sem).wait()

                def edge(r, c2):
                    off = cmp_off[pl.ds(o + r, 16)][0]
                    for k in range(DIM // 16):
                        sl = pl.ds(k * 16, 16)
                        acc[off, sl] = acc[off, sl] + rows_v[r, sl]
                    acc_cnt[off, :] = (acc_cnt[off, :]
                                       + jnp.full((16,), 1.0, jnp.float32))
                    return c2
                lax.fori_loop(0, EB, edge, 0)
                return c
            lax.fori_loop(0, nb, batch, 0)
            return c1
        lax.fori_loop(0, nseg, seg_body, 0)

        pltpu.sync_copy(acc.at[pl.ds(0, BUCK)], agg_hbm.at[pl.ds(lo, BUCK)])
        pltpu.sync_copy(acc_cnt.at[pl.ds(0, BUCK)],
                        cnt_hbm.at[pl.ds(lo, BUCK)])
        return c0
    lax.fori_loop(0, PBUCK, bucket, 0)


def _segment_sum(h, lsrc, ldst, lens):
    mesh = plsc.VectorSubcoreMesh(core_axis_name="c", subcore_axis_name="s")
    kf = pl.kernel(
        _k3b_body,
        out_type=(jax.ShapeDtypeStruct((NP, DIM), jnp.float32),
                  jax.ShapeDtypeStruct((NP, 16), jnp.float32)),
        mesh=mesh,
        scratch_types=[
            pltpu.VMEM((SEG,), jnp.int32),
            pltpu.VMEM((SEG,), jnp.int32),
            pltpu.VMEM((SEG + EB + 32,), jnp.int32),
            pltpu.VMEM((SEG + EB + 32,), jnp.int32),
            pltpu.VMEM((EB, DIM), jnp.float32),
            pltpu.VMEM((16,), jnp.int32),
            pltpu.VMEM((ACCR, DIM), jnp.float32),
            pltpu.VMEM((ACCR, 16), jnp.float32),
            pltpu.SemaphoreType.DMA,
        ],
        compiler_params=pltpu.CompilerParams(needs_layout_passes=False),
    )
    return kf(h, lsrc, ldst, lens)


KBLK = 512


def _k4_body(agg_ref, h_ref, cnt_ref, wl_ref, wr_ref, bias_ref, o_ref):
    i = pl.program_id(0)
    inv = 1.0 / jnp.maximum(cnt_ref[:, 0:1], 1.0)
    mean = agg_ref[...] * inv
    out = (jnp.dot(mean, wl_ref[...], preferred_element_type=jnp.float32)
           + jnp.dot(h_ref[...], wr_ref[...], preferred_element_type=jnp.float32)
           + bias_ref[...])
    rows = i * KBLK + lax.broadcasted_iota(jnp.int32, (KBLK, DIM), 0)
    o_ref[...] = jnp.where(rows < NFACE, out, 0.0)


def _dense(agg, h, cnt, wl_t, wr_t, bias):
    return pl.pallas_call(
        _k4_body,
        grid=(NP // KBLK,),
        in_specs=[
            pl.BlockSpec((KBLK, DIM), lambda i: (i, 0)),
            pl.BlockSpec((KBLK, DIM), lambda i: (i, 0)),
            pl.BlockSpec((KBLK, 16), lambda i: (i, 0)),
            pl.BlockSpec((DIM, DIM), lambda i: (0, 0)),
            pl.BlockSpec((DIM, DIM), lambda i: (0, 0)),
            pl.BlockSpec((1, DIM), lambda i: (0, 0)),
        ],
        out_specs=pl.BlockSpec((KBLK, DIM), lambda i: (i, 0)),
        out_shape=jax.ShapeDtypeStruct((NP, DIM), jnp.float32),
    )(agg, h, cnt, wl_t, wr_t, bias)


def kernel(vertices, faces, face_edges, face_edges_mask, coor_embed,
           W_in, b_in, Wl0, Wr0, bl0, Wl1, Wr1, bl1):
    codes = []
    for c in range(3):
        cc = vertices[:, :, c].reshape(BN)
        cc = jnp.pad(cc, (0, BNp - BN))
        codes.append(cc.reshape(BNp // VBLK, 1, VBLK))
    vert_off = jnp.repeat(jnp.arange(B, dtype=jnp.int32) * NV, NF)
    fidx = []
    for s in range(3):
        fs = faces[:, :, s].reshape(NFACE) + vert_off + s * BNp
        fs = jnp.pad(fs, (0, NP - NFACE), constant_values=s * BNp)
        fidx.append(fs)

    tables = _fuse_tables(coor_embed, W_in)
    u = _vertex_embed(codes, tables)
    u_flat = u.reshape(3 * BNp, DIM)
    x = _face_assemble(u_flat, fidx, b_in)

    eoff = (jnp.arange(B, dtype=jnp.int32) * NF)[:, None, None]
    edges = face_edges + eoff
    src = jnp.pad(edges[:, 0, :].reshape(B * E), (0, EP - B * E),
                  constant_values=NFACE)
    dst = jnp.pad(edges[:, 1, :].reshape(B * E), (0, EP - B * E),
                  constant_values=NFACE)

    lsrc, ldst, lens = _partition(src, dst)
    agg0, cnt = _segment_sum(x, lsrc, ldst, lens)
    h1 = _dense(agg0, x, cnt, Wl0.T, Wr0.T, bl0.reshape(1, DIM))
    agg1, _ = _segment_sum(h1, lsrc, ldst, lens)
    h2 = _dense(agg1, h1, cnt, Wl1.T, Wr1.T, bl1.reshape(1, DIM))
    return h2[:NFACE].reshape(B, NF, DIM)
